# Initial kernel scaffold; baseline (speedup 1.0000x reference)
import jax
import jax.numpy as jnp
from jax.experimental import pallas as pl


def kernel(nodes, neighbors, centrality, walk_choices, mask_rand):
    raise NotImplementedError("write your pallas kernel here")


# speedup vs baseline: 53.7553x; 53.7553x over previous
import functools

import jax
import jax.numpy as jnp
from jax import lax
from jax.experimental import pallas as pl
from jax.experimental.pallas import tpu as pltpu
from jax.experimental.pallas import tpu_sc as plsc

N_PATH = 16
K_PATH = 4
L_PATH = 8
DEG = 16

NT = 64
W = NT * N_PATH
NWORKERS = 32


def _sampler_body(n_node, ntiles, nper,
                  nodes_h, neigh_h, cent_h, choices_h, mr_h, out_h,
                  cent_v, nodes_v, choices_v, mr_v, idx_v, nxt_v,
                  path_v, score_v, out_v, sem):
    wid = lax.axis_index("s") * 2 + lax.axis_index("c")
    pltpu.sync_copy(cent_h, cent_v)

    iota = lax.iota(jnp.int32, 16)
    iota7 = iota * 7
    iota8w = (iota & 7) * W
    lane_lo = iota < 8
    neg_inf = jnp.float32(float("-inf"))

    def do_tile(tid):
        base = jnp.minimum(tid * NT, n_node - NT)
        wbase = base * N_PATH
        pltpu.sync_copy(nodes_h.at[pl.ds(base, NT)], nodes_v)
        pltpu.sync_copy(choices_h.at[pl.ds(wbase * (L_PATH - 1),
                                           W * (L_PATH - 1))], choices_v)
        pltpu.sync_copy(mr_h.at[pl.ds(wbase, W)], mr_v)

        @pl.loop(0, NT)
        def _init(v):
            o = v * N_PATH
            start = plsc.load_gather(nodes_v, [jnp.zeros((16,), jnp.int32) + v])
            path_v[pl.ds(o, 16)] = start
            score_v[pl.ds(o, 16)] = plsc.load_gather(cent_v, [start])
            ch = plsc.load_gather(choices_v, [iota7 + v * (16 * (L_PATH - 1))])
            idx_v[pl.ds(o, 16)] = start * DEG + ch

        for t in range(1, L_PATH):
            pltpu.async_copy(neigh_h.at[idx_v], nxt_v, sem).wait()

            @pl.loop(0, NT)
            def _step(v):
                o = v * N_PATH
                nxt = nxt_v[pl.ds(o, 16)]
                mr = mr_v[pl.ds(o, 16)]
                keep = mr >= t
                path_v[pl.ds(t * W + o, 16)] = jnp.where(keep, nxt, -1)
                c = plsc.load_gather(cent_v, [nxt])
                score_v[pl.ds(o, 16)] = score_v[pl.ds(o, 16)] + jnp.where(
                    keep, c, jnp.float32(0.0))
                if t < L_PATH - 1:
                    ch = plsc.load_gather(
                        choices_v, [iota7 + (v * (16 * (L_PATH - 1)) + t)])
                    idx_v[pl.ds(o, 16)] = nxt * DEG + ch

        @pl.loop(0, NT)
        def _select(v):
            o = v * N_PATH
            s = score_v[pl.ds(o, 16)]
            picks = []
            for _ in range(K_PATH):
                m = jnp.max(s)
                i = plsc.all_reduce_ffs(s == m)
                picks.append(i)
                s = jnp.where(iota == i, neg_inf, s)
            sel_a = jnp.where(lane_lo, picks[0], picks[1])
            pa = plsc.load_gather(path_v, [iota8w + o + sel_a])
            out_v[pl.ds(v * 32, 16)] = pa
            sel_b = jnp.where(lane_lo, picks[2], picks[3])
            pb = plsc.load_gather(path_v, [iota8w + o + sel_b])
            out_v[pl.ds(v * 32 + 16, 16)] = pb

        pltpu.sync_copy(out_v, out_h.at[pl.ds(base * 32, NT * 32)])

    @pl.loop(0, nper)
    def _tiles(j):
        tid = wid + j * NWORKERS

        @pl.when(tid < ntiles)
        def _():
            do_tile(tid)


def kernel(nodes, neighbors, centrality, walk_choices, mask_rand):
    n_node = nodes.shape[0]
    ntiles = -(-n_node // NT)
    nper = -(-ntiles // NWORKERS)

    mesh = plsc.VectorSubcoreMesh(core_axis_name="c", subcore_axis_name="s")
    run = pl.kernel(
        functools.partial(_sampler_body, n_node, ntiles, nper),
        out_type=jax.ShapeDtypeStruct((n_node * K_PATH * L_PATH,), jnp.int32),
        mesh=mesh,
        compiler_params=pltpu.CompilerParams(needs_layout_passes=False),
        scratch_types=[
            pltpu.VMEM((n_node,), jnp.float32),
            pltpu.VMEM((NT,), jnp.int32),
            pltpu.VMEM((W * (L_PATH - 1),), jnp.int32),
            pltpu.VMEM((W,), jnp.int32),
            pltpu.VMEM((W,), jnp.int32),
            pltpu.VMEM((W,), jnp.int32),
            pltpu.VMEM((L_PATH * W,), jnp.int32),
            pltpu.VMEM((W,), jnp.float32),
            pltpu.VMEM((NT * K_PATH * L_PATH,), jnp.int32),
            pltpu.SemaphoreType.DMA,
        ],
    )
    out = run(nodes,
              neighbors.reshape(-1),
              centrality,
              walk_choices.reshape(-1),
              mask_rand.reshape(-1))
    return out.reshape(n_node, K_PATH, L_PATH)


# speedup vs baseline: 67.3781x; 1.2534x over previous
import functools

import jax
import jax.numpy as jnp
from jax import lax
from jax.experimental import pallas as pl
from jax.experimental.pallas import tpu as pltpu
from jax.experimental.pallas import tpu_sc as plsc

N_PATH = 16
K_PATH = 4
L_PATH = 8
DEG = 16

NT = 128
HN = NT // 2
W = NT * N_PATH
HW = HN * N_PATH
NWORKERS = 32


def _sampler_body(n_node, ntiles, nper,
                  nodes_h, neigh_h, cent_h, choices_h, mr_h, out_h,
                  cent_v, nodes_v, choices_v, mr_v,
                  idx_a, idx_b, nxt_a, nxt_b,
                  path_v, score_v, out_v, sem_a, sem_b):
    wid = lax.axis_index("s") * 2 + lax.axis_index("c")
    pltpu.sync_copy(cent_h, cent_v)

    iota = lax.iota(jnp.int32, 16)
    iota7 = iota * (L_PATH - 1)
    iota8w = (iota & 7) * W
    lane_lo = iota < 8
    neg_inf = jnp.float32(float("-inf"))

    def init_half(lo, idx_ref):
        @pl.loop(lo, lo + HN)
        def _init(v):
            o = v * N_PATH
            ol = o - lo * N_PATH
            start = plsc.load_gather(nodes_v, [jnp.zeros((16,), jnp.int32) + v])
            path_v[pl.ds(o, 16)] = start
            score_v[pl.ds(o, 16)] = plsc.load_gather(cent_v, [start])
            ch = plsc.load_gather(choices_v,
                                  [iota7 + v * (N_PATH * (L_PATH - 1))])
            idx_ref[pl.ds(ol, 16)] = start * DEG + ch

    def process_half(lo, nxt_ref, idx_ref, t):
        @pl.loop(lo, lo + HN)
        def _step(v):
            o = v * N_PATH
            ol = o - lo * N_PATH
            nxt = nxt_ref[pl.ds(ol, 16)]
            mr = mr_v[pl.ds(o, 16)]
            keep = mr >= t
            path_v[pl.ds(t * W + o, 16)] = jnp.where(keep, nxt, -1)
            c = plsc.load_gather(cent_v, [nxt])
            score_v[pl.ds(o, 16)] = score_v[pl.ds(o, 16)] + jnp.where(
                keep, c, jnp.float32(0.0))
            if t < L_PATH - 1:
                ch = plsc.load_gather(
                    choices_v, [iota7 + (v * (N_PATH * (L_PATH - 1)) + t)])
                idx_ref[pl.ds(ol, 16)] = nxt * DEG + ch

    def do_tile(tid):
        base = jnp.minimum(tid * NT, n_node - NT)
        wbase = base * N_PATH
        pltpu.sync_copy(nodes_h.at[pl.ds(base, NT)], nodes_v)
        pltpu.sync_copy(choices_h.at[pl.ds(wbase * (L_PATH - 1),
                                           W * (L_PATH - 1))], choices_v)
        pltpu.sync_copy(mr_h.at[pl.ds(wbase, W)], mr_v)

        init_half(0, idx_a)
        cp_a = pltpu.async_copy(neigh_h.at[idx_a], nxt_a, sem_a)
        init_half(HN, idx_b)
        cp_b = pltpu.async_copy(neigh_h.at[idx_b], nxt_b, sem_b)

        for t in range(1, L_PATH):
            cp_a.wait()
            process_half(0, nxt_a, idx_a, t)
            if t < L_PATH - 1:
                cp_a = pltpu.async_copy(neigh_h.at[idx_a], nxt_a, sem_a)
            cp_b.wait()
            process_half(HN, nxt_b, idx_b, t)
            if t < L_PATH - 1:
                cp_b = pltpu.async_copy(neigh_h.at[idx_b], nxt_b, sem_b)

        @pl.loop(0, NT)
        def _select(v):
            o = v * N_PATH
            s = score_v[pl.ds(o, 16)]
            picks = []
            for _ in range(K_PATH):
                m = jnp.max(s)
                i = plsc.all_reduce_ffs(s == m)
                picks.append(i)
                s = jnp.where(iota == i, neg_inf, s)
            sel_a = jnp.where(lane_lo, picks[0], picks[1])
            pa = plsc.load_gather(path_v, [iota8w + o + sel_a])
            out_v[pl.ds(v * 32, 16)] = pa
            sel_b = jnp.where(lane_lo, picks[2], picks[3])
            pb = plsc.load_gather(path_v, [iota8w + o + sel_b])
            out_v[pl.ds(v * 32 + 16, 16)] = pb

        pltpu.sync_copy(out_v, out_h.at[pl.ds(base * 32, NT * 32)])

    @pl.loop(0, nper)
    def _tiles(j):
        tid = wid + j * NWORKERS

        @pl.when(tid < ntiles)
        def _():
            do_tile(tid)


def kernel(nodes, neighbors, centrality, walk_choices, mask_rand):
    n_node = nodes.shape[0]
    ntiles = -(-n_node // NT)
    nper = -(-ntiles // NWORKERS)

    mesh = plsc.VectorSubcoreMesh(core_axis_name="c", subcore_axis_name="s")
    run = pl.kernel(
        functools.partial(_sampler_body, n_node, ntiles, nper),
        out_type=jax.ShapeDtypeStruct((n_node * K_PATH * L_PATH,), jnp.int32),
        mesh=mesh,
        compiler_params=pltpu.CompilerParams(needs_layout_passes=False),
        scratch_types=[
            pltpu.VMEM((n_node,), jnp.float32),
            pltpu.VMEM((NT,), jnp.int32),
            pltpu.VMEM((W * (L_PATH - 1),), jnp.int32),
            pltpu.VMEM((W,), jnp.int32),
            pltpu.VMEM((HW,), jnp.int32),
            pltpu.VMEM((HW,), jnp.int32),
            pltpu.VMEM((HW,), jnp.int32),
            pltpu.VMEM((HW,), jnp.int32),
            pltpu.VMEM((L_PATH * W,), jnp.int32),
            pltpu.VMEM((W,), jnp.float32),
            pltpu.VMEM((NT * K_PATH * L_PATH,), jnp.int32),
            pltpu.SemaphoreType.DMA,
            pltpu.SemaphoreType.DMA,
        ],
    )
    out = run(nodes,
              neighbors.reshape(-1),
              centrality,
              walk_choices.reshape(-1),
              mask_rand.reshape(-1))
    return out.reshape(n_node, K_PATH, L_PATH)


# speedup vs baseline: 76.6924x; 1.1382x over previous
import functools

import jax
import jax.numpy as jnp
from jax import lax
from jax.experimental import pallas as pl
from jax.experimental.pallas import tpu as pltpu
from jax.experimental.pallas import tpu_sc as plsc

N_PATH = 16
K_PATH = 4
L_PATH = 8
DEG = 16

NT = 128
HN = NT // 2
W = NT * N_PATH
HW = HN * N_PATH
NWORKERS = 32


def _sampler_body(n_node, ntiles, nper,
                  nodes_h, neigh_h, cent_h, choices_h, mr_h, out_h,
                  cent_v, nodes_v, choices_v, mr_v,
                  idx_a, idx_b, nxt_a, nxt_b,
                  path_v, score_v, out_v, sem_a, sem_b, sem_c):
    wid = lax.axis_index("s") * 2 + lax.axis_index("c")
    pltpu.sync_copy(cent_h, cent_v)

    iota = lax.iota(jnp.int32, 16)
    iota8w = (iota & 7) * W
    lane_lo = iota < 8
    kofn_a = jnp.where(lane_lo, 0, 8 * NT) + (iota & 7) * NT
    kofn_b = kofn_a + 16 * NT
    neg_inf = jnp.float32(float("-inf"))
    nwalks = n_node * N_PATH

    def init_half(lo, idx_ref):
        @pl.loop(lo, lo + HN)
        def _init(v):
            o = v * N_PATH
            ol = o - lo * N_PATH
            start = plsc.load_gather(nodes_v, [jnp.zeros((16,), jnp.int32) + v])
            path_v[pl.ds(o, 16)] = start
            score_v[pl.ds(o, 16)] = plsc.load_gather(cent_v, [start])
            ch = choices_v[pl.ds(o, 16)]
            idx_ref[pl.ds(ol, 16)] = ch * n_node + start

    def process_half(lo, nxt_ref, idx_ref, t):
        @pl.loop(lo, lo + HN)
        def _step(v):
            o = v * N_PATH
            ol = o - lo * N_PATH
            nxt = nxt_ref[pl.ds(ol, 16)]
            mr = mr_v[pl.ds(o, 16)]
            keep = mr >= t
            path_v[pl.ds(t * W + o, 16)] = jnp.where(keep, nxt, -1)
            c = plsc.load_gather(cent_v, [nxt])
            score_v[pl.ds(o, 16)] = score_v[pl.ds(o, 16)] + jnp.where(
                keep, c, jnp.float32(0.0))
            if t < L_PATH - 1:
                ch = choices_v[pl.ds(t * W + o, 16)]
                idx_ref[pl.ds(ol, 16)] = ch * n_node + nxt

    def do_tile(tid):
        base = jnp.minimum(tid * NT, n_node - NT)
        wbase = base * N_PATH
        pltpu.sync_copy(nodes_h.at[pl.ds(base, NT)], nodes_v)
        pltpu.sync_copy(mr_h.at[pl.ds(wbase, W)], mr_v)
        ch_cps = [
            pltpu.async_copy(choices_h.at[pl.ds(t * nwalks + wbase, W)],
                             choices_v.at[pl.ds(t * W, W)], sem_c)
            for t in range(L_PATH - 1)
        ]
        for cp in ch_cps:
            cp.wait()

        init_half(0, idx_a)
        cp_a = pltpu.async_copy(neigh_h.at[idx_a], nxt_a, sem_a)
        init_half(HN, idx_b)
        cp_b = pltpu.async_copy(neigh_h.at[idx_b], nxt_b, sem_b)

        for t in range(1, L_PATH):
            cp_a.wait()
            process_half(0, nxt_a, idx_a, t)
            if t < L_PATH - 1:
                cp_a = pltpu.async_copy(neigh_h.at[idx_a], nxt_a, sem_a)
            cp_b.wait()
            process_half(HN, nxt_b, idx_b, t)
            if t < L_PATH - 1:
                cp_b = pltpu.async_copy(neigh_h.at[idx_b], nxt_b, sem_b)

        @pl.loop(0, NT)
        def _select(v):
            o = v * N_PATH
            s = score_v[pl.ds(o, 16)]
            picks = []
            for _ in range(K_PATH):
                m = jnp.max(s)
                i = plsc.all_reduce_ffs(s == m)
                picks.append(i)
                s = jnp.where(iota == i, neg_inf, s)
            sel_a = jnp.where(lane_lo, picks[0], picks[1])
            pa = plsc.load_gather(path_v, [iota8w + o + sel_a])
            plsc.store_scatter(out_v, [kofn_a + v], pa)
            sel_b = jnp.where(lane_lo, picks[2], picks[3])
            pb = plsc.load_gather(path_v, [iota8w + o + sel_b])
            plsc.store_scatter(out_v, [kofn_b + v], pb)

        out_cps = [
            pltpu.async_copy(out_v.at[pl.ds(seg * NT, NT)],
                             out_h.at[pl.ds(seg * n_node + base, NT)], sem_c)
            for seg in range(K_PATH * L_PATH)
        ]
        for cp in out_cps:
            cp.wait()

    @pl.loop(0, nper)
    def _tiles(j):
        tid = wid + j * NWORKERS

        @pl.when(tid < ntiles)
        def _():
            do_tile(tid)


def kernel(nodes, neighbors, centrality, walk_choices, mask_rand):
    n_node = nodes.shape[0]
    ntiles = -(-n_node // NT)
    nper = -(-ntiles // NWORKERS)

    mesh = plsc.VectorSubcoreMesh(core_axis_name="c", subcore_axis_name="s")
    run = pl.kernel(
        functools.partial(_sampler_body, n_node, ntiles, nper),
        out_type=jax.ShapeDtypeStruct((n_node * K_PATH * L_PATH,), jnp.int32),
        mesh=mesh,
        compiler_params=pltpu.CompilerParams(needs_layout_passes=False),
        scratch_types=[
            pltpu.VMEM((n_node,), jnp.float32),
            pltpu.VMEM((NT,), jnp.int32),
            pltpu.VMEM(((L_PATH - 1) * W,), jnp.int32),
            pltpu.VMEM((W,), jnp.int32),
            pltpu.VMEM((HW,), jnp.int32),
            pltpu.VMEM((HW,), jnp.int32),
            pltpu.VMEM((HW,), jnp.int32),
            pltpu.VMEM((HW,), jnp.int32),
            pltpu.VMEM((L_PATH * W,), jnp.int32),
            pltpu.VMEM((W,), jnp.float32),
            pltpu.VMEM((K_PATH * L_PATH * NT,), jnp.int32),
            pltpu.SemaphoreType.DMA,
            pltpu.SemaphoreType.DMA,
            pltpu.SemaphoreType.DMA,
        ],
    )
    out = run(nodes,
              jnp.swapaxes(neighbors, 0, 1).reshape(-1),
              centrality,
              jnp.swapaxes(walk_choices, 0, 1).reshape(-1),
              mask_rand.reshape(-1))
    return jnp.transpose(out.reshape(K_PATH, L_PATH, n_node), (2, 0, 1))


# speedup vs baseline: 100.1546x; 1.3059x over previous
import functools

import jax
import jax.numpy as jnp
from jax import lax
from jax.experimental import pallas as pl
from jax.experimental.pallas import tpu as pltpu
from jax.experimental.pallas import tpu_sc as plsc

def _vshuf(x, idx):
    return lax.gather(
        x, idx[:, None],
        dimension_numbers=lax.GatherDimensionNumbers(
            offset_dims=(), collapsed_slice_dims=(0,), start_index_map=(0,)),
        slice_sizes=(1,), mode=lax.GatherScatterMode.PROMISE_IN_BOUNDS)


N_PATH = 16
K_PATH = 4
L_PATH = 8
DEG = 16

NT = 128
HN = NT // 2
W = NT * N_PATH
HW = HN * N_PATH
NWORKERS = 32


def _sampler_body(n_node, ntiles, nper,
                  nodes_h, neigh_h, cent_h,
                  ch0_h, ch1_h, ch2_h, ch3_h, ch4_h, ch5_h, ch6_h,
                  mr_h, out_h,
                  cent_v, nodes_v, choices_v, mr_v,
                  idx_a, idx_b, nxt_a, nxt_b,
                  path_v, score_v, out_v, sem_a, sem_b, sem_c):
    wid = lax.axis_index("s") * 2 + lax.axis_index("c")
    pltpu.sync_copy(cent_h, cent_v)

    iota = lax.iota(jnp.int32, 16)
    iota8w = (iota & 7) * W
    lane_lo = iota < 8
    kofn_a = jnp.where(lane_lo, 0, 8 * NT) + (iota & 7) * NT
    kofn_b = kofn_a + 16 * NT
    choices_hs = (ch0_h, ch1_h, ch2_h, ch3_h, ch4_h, ch5_h, ch6_h)
    iota_nt = lax.iota(jnp.int32, 16) * NT
    neg_inf = jnp.float32(float("-inf"))
    def init_half(lo, idx_ref):
        @pl.loop(lo, lo + HN, unroll=4)
        def _init(v):
            o = v * N_PATH
            ol = o - lo * N_PATH
            start = plsc.load_gather(nodes_v, [jnp.zeros((16,), jnp.int32) + v])
            path_v[pl.ds(o, 16)] = start
            score_v[pl.ds(o, 16)] = plsc.load_gather(cent_v, [start])
            ch = choices_v[pl.ds(o, 16)]
            idx_ref[pl.ds(ol, 16)] = ch * n_node + start

    def process_half(lo, nxt_ref, idx_ref, t):
        @pl.loop(lo, lo + HN, unroll=4)
        def _step(v):
            o = v * N_PATH
            ol = o - lo * N_PATH
            nxt = nxt_ref[pl.ds(ol, 16)]
            mr = plsc.load_gather(mr_v, [iota_nt + v])
            keep = mr >= t
            path_v[pl.ds(t * W + o, 16)] = jnp.where(keep, nxt, -1)
            c = plsc.load_gather(cent_v, [nxt])
            score_v[pl.ds(o, 16)] = score_v[pl.ds(o, 16)] + jnp.where(
                keep, c, jnp.float32(0.0))
            if t < L_PATH - 1:
                ch = choices_v[pl.ds(t * W + o, 16)]
                idx_ref[pl.ds(ol, 16)] = ch * n_node + nxt

    def do_tile(tid):
        base = jnp.minimum(tid * NT, n_node - NT)
        wbase = base * N_PATH
        pltpu.sync_copy(nodes_h.at[pl.ds(base, NT)], nodes_v)
        ch_cps = [
            pltpu.async_copy(choices_hs[t].at[pl.ds(wbase, W)],
                             choices_v.at[pl.ds(t * W, W)], sem_c)
            for t in range(L_PATH - 1)
        ] + [
            pltpu.async_copy(mr_h.at[pl.ds(p * n_node + base, NT)],
                             mr_v.at[pl.ds(p * NT, NT)], sem_c)
            for p in range(N_PATH)
        ]
        for cp in ch_cps:
            cp.wait()

        init_half(0, idx_a)
        cp_a = pltpu.async_copy(neigh_h.at[idx_a], nxt_a, sem_a)
        init_half(HN, idx_b)
        cp_b = pltpu.async_copy(neigh_h.at[idx_b], nxt_b, sem_b)

        for t in range(1, L_PATH):
            cp_a.wait()
            process_half(0, nxt_a, idx_a, t)
            if t < L_PATH - 1:
                cp_a = pltpu.async_copy(neigh_h.at[idx_a], nxt_a, sem_a)
            cp_b.wait()
            process_half(HN, nxt_b, idx_b, t)
            if t < L_PATH - 1:
                cp_b = pltpu.async_copy(neigh_h.at[idx_b], nxt_b, sem_b)

        @pl.loop(0, NT)
        def _select(v):
            o = v * N_PATH
            s = score_v[pl.ds(o, 16)]
            picks = []
            for _ in range(K_PATH):
                m = s
                for sh in (1, 2, 4, 8):
                    m = jnp.maximum(m, _vshuf(m, iota ^ sh))
                i = plsc.all_reduce_ffs(s == m)
                picks.append(i)
                s = jnp.where(iota == i, neg_inf, s)
            sel_a = jnp.where(lane_lo, picks[0], picks[1])
            pa = plsc.load_gather(path_v, [iota8w + o + sel_a])
            plsc.store_scatter(out_v, [kofn_a + v], pa)
            sel_b = jnp.where(lane_lo, picks[2], picks[3])
            pb = plsc.load_gather(path_v, [iota8w + o + sel_b])
            plsc.store_scatter(out_v, [kofn_b + v], pb)

        out_cps = [
            pltpu.async_copy(out_v.at[pl.ds(seg * NT, NT)],
                             out_h.at[pl.ds(seg * n_node + base, NT)], sem_c)
            for seg in range(K_PATH * L_PATH)
        ]
        for cp in out_cps:
            cp.wait()

    @pl.loop(0, nper)
    def _tiles(j):
        tid = wid + j * NWORKERS

        @pl.when(tid < ntiles)
        def _():
            do_tile(tid)


def kernel(nodes, neighbors, centrality, walk_choices, mask_rand):
    n_node = nodes.shape[0]
    ntiles = -(-n_node // NT)
    nper = -(-ntiles // NWORKERS)

    mesh = plsc.VectorSubcoreMesh(core_axis_name="c", subcore_axis_name="s")
    run = pl.kernel(
        functools.partial(_sampler_body, n_node, ntiles, nper),
        out_type=jax.ShapeDtypeStruct((n_node * K_PATH * L_PATH,), jnp.int32),
        mesh=mesh,
        compiler_params=pltpu.CompilerParams(needs_layout_passes=False),
        scratch_types=[
            pltpu.VMEM((n_node,), jnp.float32),
            pltpu.VMEM((NT,), jnp.int32),
            pltpu.VMEM(((L_PATH - 1) * W,), jnp.int32),
            pltpu.VMEM((W,), jnp.int32),
            pltpu.VMEM((HW,), jnp.int32),
            pltpu.VMEM((HW,), jnp.int32),
            pltpu.VMEM((HW,), jnp.int32),
            pltpu.VMEM((HW,), jnp.int32),
            pltpu.VMEM((L_PATH * W,), jnp.int32),
            pltpu.VMEM((W,), jnp.float32),
            pltpu.VMEM((K_PATH * L_PATH * NT,), jnp.int32),
            pltpu.SemaphoreType.DMA,
            pltpu.SemaphoreType.DMA,
            pltpu.SemaphoreType.DMA,
        ],
    )
    cols = [walk_choices[:, t] for t in range(L_PATH - 1)]
    out = run(nodes,
              jnp.swapaxes(neighbors, 0, 1).reshape(-1),
              centrality,
              *cols,
              jnp.swapaxes(mask_rand, 0, 1).reshape(-1))
    return jnp.transpose(out.reshape(K_PATH, L_PATH, n_node), (2, 0, 1))


# speedup vs baseline: 121.6759x; 1.2149x over previous
import functools

import jax
import jax.numpy as jnp
from jax import lax
from jax.experimental import pallas as pl
from jax.experimental.pallas import tpu as pltpu
from jax.experimental.pallas import tpu_sc as plsc

def _vshuf(x, idx):
    return lax.gather(
        x, idx[:, None],
        dimension_numbers=lax.GatherDimensionNumbers(
            offset_dims=(), collapsed_slice_dims=(0,), start_index_map=(0,)),
        slice_sizes=(1,), mode=lax.GatherScatterMode.PROMISE_IN_BOUNDS)


N_PATH = 16
K_PATH = 4
L_PATH = 8
DEG = 16

NT = 128
HN = NT // 2
W = NT * N_PATH
HW = HN * N_PATH
NWORKERS = 32


def _sampler_body(n_node, ntiles, nper,
                  nodes_h, neigh_h, cent_h, choices_h, mr_h, out_h,
                  cent_v, nodes_v, choices_v, mr_v,
                  idx_a, idx_b, nxt_a, nxt_b,
                  path_v, score_v, out_v, sem_a, sem_b, sem_c):
    wid = lax.axis_index("s") * 2 + lax.axis_index("c")
    pltpu.sync_copy(cent_h, cent_v)

    iota = lax.iota(jnp.int32, 16)
    iota8w = (iota & 7) * W
    lane_lo = iota < 8
    kofn_a = jnp.where(lane_lo, 0, 8 * NT) + (iota & 7) * NT
    kofn_b = kofn_a + 16 * NT
    iota16 = lax.iota(jnp.int32, 16)
    neg_inf = jnp.float32(float("-inf"))
    MRW = 256

    def init_half(lo, idx_ref):
        @pl.loop(lo, lo + HN)
        def _init(v):
            o = v * N_PATH
            ol = o - lo * N_PATH
            start = plsc.load_gather(nodes_v, [jnp.zeros((16,), jnp.int32) + v])
            path_v[pl.ds(o, 16)] = start
            score_v[pl.ds(o, 16)] = plsc.load_gather(cent_v, [start])
            ch = choices_v[0, pl.ds(o, 16)]
            idx_ref[pl.ds(ol, 16)] = ch * n_node + start

    def process_half(lo, nxt_ref, idx_ref, t, off):
        @pl.loop(lo, lo + HN)
        def _step(v):
            o = v * N_PATH
            ol = o - lo * N_PATH
            nxt = nxt_ref[pl.ds(ol, 16)]
            mr = plsc.load_gather(mr_v, [iota16, off + v])
            keep = mr >= t
            path_v[pl.ds(t * W + o, 16)] = jnp.where(keep, nxt, -1)
            c = plsc.load_gather(cent_v, [nxt])
            score_v[pl.ds(o, 16)] = score_v[pl.ds(o, 16)] + jnp.where(
                keep, c, jnp.float32(0.0))
            if t < L_PATH - 1:
                ch = choices_v[t, pl.ds(o, 16)]
                idx_ref[pl.ds(ol, 16)] = ch * n_node + nxt

    def do_tile(tid):
        base = jnp.minimum(tid * NT, n_node - NT)
        wbase = pl.multiple_of(base * N_PATH, 128)
        base0 = pl.multiple_of(base - lax.rem(base, 128), 128)
        off = (base - base0) + jnp.zeros((16,), jnp.int32)
        pltpu.sync_copy(nodes_h.at[pl.ds(base, NT)], nodes_v)
        cp_ch = pltpu.async_copy(choices_h.at[:, pl.ds(wbase, W)],
                                 choices_v, sem_c)
        cp_mr = pltpu.async_copy(mr_h.at[:, pl.ds(base0, MRW)], mr_v, sem_c)
        cp_ch.wait()
        cp_mr.wait()

        init_half(0, idx_a)
        cp_a = pltpu.async_copy(neigh_h.at[idx_a], nxt_a, sem_a)
        init_half(HN, idx_b)
        cp_b = pltpu.async_copy(neigh_h.at[idx_b], nxt_b, sem_b)

        for t in range(1, L_PATH):
            cp_a.wait()
            process_half(0, nxt_a, idx_a, t, off)
            if t < L_PATH - 1:
                cp_a = pltpu.async_copy(neigh_h.at[idx_a], nxt_a, sem_a)
            cp_b.wait()
            process_half(HN, nxt_b, idx_b, t, off)
            if t < L_PATH - 1:
                cp_b = pltpu.async_copy(neigh_h.at[idx_b], nxt_b, sem_b)

        @pl.loop(0, NT)
        def _select(v):
            o = v * N_PATH
            s = score_v[pl.ds(o, 16)]
            picks = []
            for _ in range(K_PATH):
                m = s
                for sh in (1, 2, 4, 8):
                    m = jnp.maximum(m, _vshuf(m, iota ^ sh))
                i = plsc.all_reduce_ffs(s == m)
                picks.append(i)
                s = jnp.where(iota == i, neg_inf, s)
            sel_a = jnp.where(lane_lo, picks[0], picks[1])
            pa = plsc.load_gather(path_v, [iota8w + o + sel_a])
            plsc.store_scatter(out_v, [kofn_a + v], pa)
            sel_b = jnp.where(lane_lo, picks[2], picks[3])
            pb = plsc.load_gather(path_v, [iota8w + o + sel_b])
            plsc.store_scatter(out_v, [kofn_b + v], pb)

        out_cps = [
            pltpu.async_copy(out_v.at[pl.ds(seg * NT, NT)],
                             out_h.at[pl.ds(seg * n_node + base, NT)], sem_c)
            for seg in range(K_PATH * L_PATH)
        ]
        for cp in out_cps:
            cp.wait()

    @pl.loop(0, nper)
    def _tiles(j):
        tid = wid + j * NWORKERS

        @pl.when(tid < ntiles)
        def _():
            do_tile(tid)


def kernel(nodes, neighbors, centrality, walk_choices, mask_rand):
    n_node = nodes.shape[0]
    ntiles = -(-n_node // NT)
    nper = -(-ntiles // NWORKERS)
    max_base0 = (n_node - NT) - ((n_node - NT) % 128)
    mr_pad = max_base0 + 256 - n_node

    mesh = plsc.VectorSubcoreMesh(core_axis_name="c", subcore_axis_name="s")
    run = pl.kernel(
        functools.partial(_sampler_body, n_node, ntiles, nper),
        out_type=jax.ShapeDtypeStruct((n_node * K_PATH * L_PATH,), jnp.int32),
        mesh=mesh,
        compiler_params=pltpu.CompilerParams(needs_layout_passes=False),
        scratch_types=[
            pltpu.VMEM((n_node,), jnp.float32),
            pltpu.VMEM((NT,), jnp.int32),
            pltpu.VMEM((L_PATH - 1, W), jnp.int32),
            pltpu.VMEM((N_PATH, 256), jnp.int32),
            pltpu.VMEM((HW,), jnp.int32),
            pltpu.VMEM((HW,), jnp.int32),
            pltpu.VMEM((HW,), jnp.int32),
            pltpu.VMEM((HW,), jnp.int32),
            pltpu.VMEM((L_PATH * W,), jnp.int32),
            pltpu.VMEM((W,), jnp.float32),
            pltpu.VMEM((K_PATH * L_PATH * NT,), jnp.int32),
            pltpu.SemaphoreType.DMA,
            pltpu.SemaphoreType.DMA,
            pltpu.SemaphoreType.DMA,
        ],
    )
    out = run(nodes,
              jnp.swapaxes(neighbors, 0, 1).reshape(-1),
              centrality,
              jnp.swapaxes(walk_choices, 0, 1),
              jnp.pad(jnp.swapaxes(mask_rand, 0, 1), ((0, 0), (0, mr_pad))))
    return jnp.transpose(out.reshape(K_PATH, L_PATH, n_node), (2, 0, 1))


# speedup vs baseline: 125.8215x; 1.0341x over previous
import functools

import jax
import jax.numpy as jnp
from jax import lax
from jax.experimental import pallas as pl
from jax.experimental.pallas import tpu as pltpu
from jax.experimental.pallas import tpu_sc as plsc

N_PATH = 16
K_PATH = 4
L_PATH = 8
DEG = 16

NT = 128
HN = NT // 2
W = NT * N_PATH
HW = HN * N_PATH
NWORKERS = 32
MRW = 256


def _sampler_body(n_node, ntiles, nper,
                  nodes_h, neigh_h, cent_h, choices_h, mr_h, out_h,
                  cent_v, nodes_v0, nodes_v1, choices_v0, choices_v1,
                  mr_v0, mr_v1, idx_a, idx_b, nxt_a, nxt_b,
                  path_v, score_v, out_v0, out_v1,
                  sem_a, sem_b, sem_c, sem_o):
    wid = lax.axis_index("s") * 2 + lax.axis_index("c")
    pltpu.sync_copy(cent_h, cent_v)

    banks = ((nodes_v0, choices_v0, mr_v0, out_v0),
             (nodes_v1, choices_v1, mr_v1, out_v1))

    iota = lax.iota(jnp.int32, 16)
    iota16 = iota
    iota8w = (iota & 7) * W
    lane_lo = iota < 8
    kofn_a = jnp.where(lane_lo, 0, 8 * NT) + (iota & 7) * NT
    kofn_b = kofn_a + 16 * NT
    neg_inf = jnp.float32(float("-inf"))

    def tile_geom(tid):
        base = tid * NT
        wbase = pl.multiple_of(base * N_PATH, 128)
        mbase = pl.multiple_of(base, 128)
        off = jnp.zeros((16,), jnp.int32)
        return base, wbase, mbase, off

    def issue_stage_at(bank, wbase, base0):
        _, choices_v, mr_v, _ = banks[bank]
        return [
            pltpu.async_copy(choices_h.at[:, pl.ds(wbase, W)], choices_v,
                             sem_c),
            pltpu.async_copy(mr_h.at[:, pl.ds(base0, MRW)], mr_v, sem_c),
        ]

    def init_half(lo, idx_ref, nodes_v, choices_v):
        @pl.loop(lo, lo + HN)
        def _init(v):
            o = v * N_PATH
            ol = o - lo * N_PATH
            start = plsc.load_gather(nodes_v, [jnp.zeros((16,), jnp.int32) + v])
            path_v[pl.ds(o, 16)] = start
            score_v[pl.ds(o, 16)] = plsc.load_gather(cent_v, [start])
            ch = choices_v[0, pl.ds(o, 16)]
            idx_ref[pl.ds(ol, 16)] = ch * n_node + start

    def process_half(lo, nxt_ref, idx_ref, t, off, choices_v, mr_v):
        @pl.loop(lo, lo + HN)
        def _step(v):
            o = v * N_PATH
            ol = o - lo * N_PATH
            nxt = nxt_ref[pl.ds(ol, 16)]
            mr = plsc.load_gather(mr_v, [iota16, off + v])
            keep = mr >= t
            path_v[pl.ds(t * W + o, 16)] = jnp.where(keep, nxt, -1)
            c = plsc.load_gather(cent_v, [nxt])
            score_v[pl.ds(o, 16)] = score_v[pl.ds(o, 16)] + jnp.where(
                keep, c, jnp.float32(0.0))
            if t < L_PATH - 1:
                ch = choices_v[t, pl.ds(o, 16)]
                idx_ref[pl.ds(ol, 16)] = ch * n_node + nxt

    def do_tile_at(bank, base, off):
        nodes_v, choices_v, mr_v, out_v = banks[bank]
        pltpu.sync_copy(nodes_h.at[pl.ds(base, NT)], nodes_v)

        init_half(0, idx_a, nodes_v, choices_v)
        cp_a = pltpu.async_copy(neigh_h.at[idx_a], nxt_a, sem_a)
        init_half(HN, idx_b, nodes_v, choices_v)
        cp_b = pltpu.async_copy(neigh_h.at[idx_b], nxt_b, sem_b)

        for t in range(1, L_PATH):
            cp_a.wait()
            process_half(0, nxt_a, idx_a, t, off, choices_v, mr_v)
            if t < L_PATH - 1:
                cp_a = pltpu.async_copy(neigh_h.at[idx_a], nxt_a, sem_a)
            cp_b.wait()
            process_half(HN, nxt_b, idx_b, t, off, choices_v, mr_v)
            if t < L_PATH - 1:
                cp_b = pltpu.async_copy(neigh_h.at[idx_b], nxt_b, sem_b)

        @pl.loop(0, NT)
        def _select(v):
            o = v * N_PATH
            s = score_v[pl.ds(o, 16)]
            picks = []
            for _ in range(K_PATH):
                m = jnp.max(s)
                i = plsc.all_reduce_ffs(s == m)
                picks.append(i)
                s = jnp.where(iota == i, neg_inf, s)
            sel_a = jnp.where(lane_lo, picks[0], picks[1])
            pa = plsc.load_gather(path_v, [iota8w + o + sel_a])
            plsc.store_scatter(out_v, [kofn_a + v], pa)
            sel_b = jnp.where(lane_lo, picks[2], picks[3])
            pb = plsc.load_gather(path_v, [iota8w + o + sel_b])
            plsc.store_scatter(out_v, [kofn_b + v], pb)

        for seg in range(K_PATH * L_PATH):
            pltpu.async_copy(out_v.at[pl.ds(seg * NT, NT)],
                             out_h.at[pl.ds(seg * n_node + base, NT)], sem_o)

    nfull = n_node // NT
    nper_full = -(-nfull // NWORKERS)

    @pl.loop(0, (nper_full + 1) // 2)
    def _pair(u):
        j0 = 2 * u
        tid0 = wid + j0 * NWORKERS
        tid1 = tid0 + NWORKERS
        @pl.when(tid0 < nfull)
        def _():
            _, w0, m0, _ = tile_geom(tid0)
            for cp in issue_stage_at(0, w0, m0):
                cp.wait()

        @pl.when(tid1 < nfull)
        def _():
            _, w1, m1, _ = tile_geom(tid1)
            issue_stage_at(1, w1, m1)

        @pl.when(tid0 < nfull)
        def _():
            b0, _, _, o0 = tile_geom(tid0)
            do_tile_at(0, b0, o0)

        @pl.when(tid1 < nfull)
        def _():
            pltpu.make_async_copy(choices_h.at[:, pl.ds(0, W)],
                                  choices_v1, sem_c).wait()
            pltpu.make_async_copy(mr_h.at[:, pl.ds(0, MRW)],
                                  mr_v1, sem_c).wait()
            b1, _, _, o1 = tile_geom(tid1)
            do_tile_at(1, b1, o1)
            pltpu.make_async_copy(out_h.at[pl.ds(0, K_PATH * L_PATH * NT)],
                                  out_v1, sem_o).wait()

        @pl.when(tid0 < nfull)
        def _():
            pltpu.make_async_copy(out_h.at[pl.ds(0, K_PATH * L_PATH * NT)],
                                  out_v0, sem_o).wait()

    if n_node % NT:
        tail_base = n_node - NT
        tail_base0 = tail_base - tail_base % 128
        tail_off = (tail_base - tail_base0) + jnp.zeros((16,), jnp.int32)

        @pl.when(wid == nfull % NWORKERS)
        def _tail():
            for cp in issue_stage_at(0, tail_base * N_PATH, tail_base0):
                cp.wait()
            do_tile_at(0, tail_base, tail_off)
            pltpu.make_async_copy(out_h.at[pl.ds(0, K_PATH * L_PATH * NT)],
                                  out_v0, sem_o).wait()


def kernel(nodes, neighbors, centrality, walk_choices, mask_rand):
    n_node = nodes.shape[0]
    ntiles = -(-n_node // NT)
    nper = -(-ntiles // NWORKERS)
    max_base0 = (n_node - NT) - ((n_node - NT) % 128)
    mr_pad = max_base0 + MRW - n_node

    mesh = plsc.VectorSubcoreMesh(core_axis_name="c", subcore_axis_name="s")
    run = pl.kernel(
        functools.partial(_sampler_body, n_node, ntiles, nper),
        out_type=jax.ShapeDtypeStruct((n_node * K_PATH * L_PATH,), jnp.int32),
        mesh=mesh,
        compiler_params=pltpu.CompilerParams(needs_layout_passes=False),
        scratch_types=[
            pltpu.VMEM((n_node,), jnp.float32),
            pltpu.VMEM((NT,), jnp.int32),
            pltpu.VMEM((NT,), jnp.int32),
            pltpu.VMEM((L_PATH - 1, W), jnp.int32),
            pltpu.VMEM((L_PATH - 1, W), jnp.int32),
            pltpu.VMEM((N_PATH, MRW), jnp.int32),
            pltpu.VMEM((N_PATH, MRW), jnp.int32),
            pltpu.VMEM((HW,), jnp.int32),
            pltpu.VMEM((HW,), jnp.int32),
            pltpu.VMEM((HW,), jnp.int32),
            pltpu.VMEM((HW,), jnp.int32),
            pltpu.VMEM((L_PATH * W,), jnp.int32),
            pltpu.VMEM((W,), jnp.float32),
            pltpu.VMEM((K_PATH * L_PATH * NT,), jnp.int32),
            pltpu.VMEM((K_PATH * L_PATH * NT,), jnp.int32),
            pltpu.SemaphoreType.DMA,
            pltpu.SemaphoreType.DMA,
            pltpu.SemaphoreType.DMA,
            pltpu.SemaphoreType.DMA,
        ],
    )
    out = run(nodes,
              jnp.swapaxes(neighbors, 0, 1).reshape(-1),
              centrality,
              jnp.swapaxes(walk_choices, 0, 1),
              jnp.pad(jnp.swapaxes(mask_rand, 0, 1), ((0, 0), (0, mr_pad))))
    return jnp.transpose(out.reshape(K_PATH, L_PATH, n_node), (2, 0, 1))


# speedup vs baseline: 127.8870x; 1.0164x over previous
import functools

import jax
import jax.numpy as jnp
from jax import lax
from jax.experimental import pallas as pl
from jax.experimental.pallas import tpu as pltpu
from jax.experimental.pallas import tpu_sc as plsc

def _vshuf(x, idx):
    return lax.gather(
        x, idx[:, None],
        dimension_numbers=lax.GatherDimensionNumbers(
            offset_dims=(), collapsed_slice_dims=(0,), start_index_map=(0,)),
        slice_sizes=(1,), mode=lax.GatherScatterMode.PROMISE_IN_BOUNDS)


N_PATH = 16
K_PATH = 4
L_PATH = 8
DEG = 16

NT = 64
HN = NT // 2
W = NT * N_PATH
HW = HN * N_PATH
NWORKERS = 32


def _sampler_body(n_node, ntiles, nper,
                  nodes_h, neigh_h, cent_h, choices_h, mr_h, out_h,
                  neigh_s, cent_v, nodes_v, choices_v, mr_v,
                  idx_a, idx_b, nxt_a, nxt_b,
                  path_v, score_v, out_v, tbl_v, sem_a, sem_b, sem_c):
    sid = lax.axis_index("s")
    wid = sid * 2 + lax.axis_index("c")

    seg = n_node * DEG // 16

    @pl.loop(0, seg // 2000)
    def _stage(ci):
        toff = sid * seg + ci * 2000
        pltpu.sync_copy(neigh_h.at[pl.ds(toff, 2000)], tbl_v)
        pltpu.sync_copy(tbl_v, neigh_s.at[pl.ds(toff, 2000)])

    pltpu.sync_copy(cent_h, cent_v)
    plsc.subcore_barrier()

    iota = lax.iota(jnp.int32, 16)
    iota8w = (iota & 7) * W
    lane_lo = iota < 8
    kofn_a = jnp.where(lane_lo, 0, 8 * NT) + (iota & 7) * NT
    kofn_b = kofn_a + 16 * NT
    iota16 = lax.iota(jnp.int32, 16)
    neg_inf = jnp.float32(float("-inf"))
    MRW = 256

    def init_half(lo, idx_ref):
        @pl.loop(lo, lo + HN)
        def _init(v):
            o = v * N_PATH
            ol = o - lo * N_PATH
            start = plsc.load_gather(nodes_v, [jnp.zeros((16,), jnp.int32) + v])
            path_v[pl.ds(o, 16)] = start
            score_v[pl.ds(o, 16)] = plsc.load_gather(cent_v, [start])
            ch = choices_v[0, pl.ds(o, 16)]
            idx_ref[pl.ds(ol, 16)] = ch * n_node + start

    def process_half(lo, nxt_ref, idx_ref, t, off):
        @pl.loop(lo, lo + HN)
        def _step(v):
            o = v * N_PATH
            ol = o - lo * N_PATH
            nxt = nxt_ref[pl.ds(ol, 16)]
            mr = plsc.load_gather(mr_v, [iota16, off + v])
            keep = mr >= t
            path_v[pl.ds(t * W + o, 16)] = jnp.where(keep, nxt, -1)
            c = plsc.load_gather(cent_v, [nxt])
            score_v[pl.ds(o, 16)] = score_v[pl.ds(o, 16)] + jnp.where(
                keep, c, jnp.float32(0.0))
            if t < L_PATH - 1:
                ch = choices_v[t, pl.ds(o, 16)]
                idx_ref[pl.ds(ol, 16)] = ch * n_node + nxt

    def do_tile(tid):
        base = jnp.minimum(tid * NT, n_node - NT)
        wbase = pl.multiple_of(base * N_PATH, 128)
        base0 = pl.multiple_of(base - lax.rem(base, 128), 128)
        off = (base - base0) + jnp.zeros((16,), jnp.int32)
        pltpu.sync_copy(nodes_h.at[pl.ds(base, NT)], nodes_v)
        cp_ch = pltpu.async_copy(choices_h.at[:, pl.ds(wbase, W)],
                                 choices_v, sem_c)
        cp_mr = pltpu.async_copy(mr_h.at[:, pl.ds(base0, MRW)], mr_v, sem_c)
        cp_ch.wait()
        cp_mr.wait()

        init_half(0, idx_a)
        cp_a = pltpu.async_copy(neigh_s.at[idx_a], nxt_a, sem_a)
        init_half(HN, idx_b)
        cp_b = pltpu.async_copy(neigh_s.at[idx_b], nxt_b, sem_b)

        for t in range(1, L_PATH):
            cp_a.wait()
            process_half(0, nxt_a, idx_a, t, off)
            if t < L_PATH - 1:
                cp_a = pltpu.async_copy(neigh_s.at[idx_a], nxt_a, sem_a)
            cp_b.wait()
            process_half(HN, nxt_b, idx_b, t, off)
            if t < L_PATH - 1:
                cp_b = pltpu.async_copy(neigh_s.at[idx_b], nxt_b, sem_b)

        @pl.loop(0, NT)
        def _select(v):
            o = v * N_PATH
            s = score_v[pl.ds(o, 16)]
            picks = []
            for _ in range(K_PATH):
                m = s
                for sh in (1, 2, 4, 8):
                    m = jnp.maximum(m, _vshuf(m, iota ^ sh))
                i = plsc.all_reduce_ffs(s == m)
                picks.append(i)
                s = jnp.where(iota == i, neg_inf, s)
            sel_a = jnp.where(lane_lo, picks[0], picks[1])
            pa = plsc.load_gather(path_v, [iota8w + o + sel_a])
            plsc.store_scatter(out_v, [kofn_a + v], pa)
            sel_b = jnp.where(lane_lo, picks[2], picks[3])
            pb = plsc.load_gather(path_v, [iota8w + o + sel_b])
            plsc.store_scatter(out_v, [kofn_b + v], pb)

        out_cps = [
            pltpu.async_copy(out_v.at[pl.ds(seg * NT, NT)],
                             out_h.at[pl.ds(seg * n_node + base, NT)], sem_c)
            for seg in range(K_PATH * L_PATH)
        ]
        for cp in out_cps:
            cp.wait()

    @pl.loop(0, nper)
    def _tiles(j):
        tid = wid + j * NWORKERS

        @pl.when(tid < ntiles)
        def _():
            do_tile(tid)


def kernel(nodes, neighbors, centrality, walk_choices, mask_rand):
    n_node = nodes.shape[0]
    ntiles = -(-n_node // NT)
    nper = -(-ntiles // NWORKERS)
    max_base0 = (n_node - NT) - ((n_node - NT) % 128)
    mr_pad = max_base0 + 256 - n_node

    mesh = plsc.VectorSubcoreMesh(core_axis_name="c", subcore_axis_name="s")
    run = pl.kernel(
        functools.partial(_sampler_body, n_node, ntiles, nper),
        out_type=jax.ShapeDtypeStruct((n_node * K_PATH * L_PATH,), jnp.int32),
        mesh=mesh,
        compiler_params=pltpu.CompilerParams(needs_layout_passes=False),
        scratch_types=[
            pltpu.VMEM_SHARED((n_node * DEG,), jnp.int32),
            pltpu.VMEM((n_node,), jnp.float32),
            pltpu.VMEM((NT,), jnp.int32),
            pltpu.VMEM((L_PATH - 1, W), jnp.int32),
            pltpu.VMEM((N_PATH, 256), jnp.int32),
            pltpu.VMEM((HW,), jnp.int32),
            pltpu.VMEM((HW,), jnp.int32),
            pltpu.VMEM((HW,), jnp.int32),
            pltpu.VMEM((HW,), jnp.int32),
            pltpu.VMEM((L_PATH * W,), jnp.int32),
            pltpu.VMEM((W,), jnp.float32),
            pltpu.VMEM((K_PATH * L_PATH * NT,), jnp.int32),
            pltpu.VMEM((2000,), jnp.int32),
            pltpu.SemaphoreType.DMA,
            pltpu.SemaphoreType.DMA,
            pltpu.SemaphoreType.DMA,
        ],
    )
    out = run(nodes,
              jnp.swapaxes(neighbors, 0, 1).reshape(-1),
              centrality,
              jnp.swapaxes(walk_choices, 0, 1),
              jnp.pad(jnp.swapaxes(mask_rand, 0, 1), ((0, 0), (0, mr_pad))))
    return jnp.transpose(out.reshape(K_PATH, L_PATH, n_node), (2, 0, 1))


# speedup vs baseline: 131.5887x; 1.0289x over previous
import functools

import jax
import jax.numpy as jnp
from jax import lax
from jax.experimental import pallas as pl
from jax.experimental.pallas import tpu as pltpu
from jax.experimental.pallas import tpu_sc as plsc

def _vshuf(x, idx):
    return lax.gather(
        x, idx[:, None],
        dimension_numbers=lax.GatherDimensionNumbers(
            offset_dims=(), collapsed_slice_dims=(0,), start_index_map=(0,)),
        slice_sizes=(1,), mode=lax.GatherScatterMode.PROMISE_IN_BOUNDS)


N_PATH = 16
K_PATH = 4
L_PATH = 8
DEG = 16

NT = 64
HN = NT // 2
W = NT * N_PATH
HW = HN * N_PATH
NWORKERS = 32


def _sampler_body(n_node, ntiles, nper,
                  nodes_h, neigh_h, cent_h, choices_h, mr_h, out_h,
                  neigh_s, cent_v, nodes_v, choices_v, mr_v,
                  idx_a, idx_b, nxt_a, nxt_b,
                  path_v, score_v, out_v, tbl_v, sem_a, sem_b, sem_c,
                  sem_o, sem_p):
    sid = lax.axis_index("s")
    wid = sid * 2 + lax.axis_index("c")

    seg = n_node * DEG // 16

    @pl.loop(0, seg // 2000)
    def _stage(ci):
        toff = sid * seg + ci * 2000
        pltpu.sync_copy(neigh_h.at[pl.ds(toff, 2000)], tbl_v)
        pltpu.sync_copy(tbl_v, neigh_s.at[pl.ds(toff, 2000)])

    pltpu.sync_copy(cent_h, cent_v)
    plsc.subcore_barrier()

    iota = lax.iota(jnp.int32, 16)
    iota8w = (iota & 7) * W
    lane_lo = iota < 8
    kofn_a = jnp.where(lane_lo, 0, 8 * NT) + (iota & 7) * NT
    kofn_b = kofn_a + 16 * NT
    iota16 = lax.iota(jnp.int32, 16)
    neg_inf = jnp.float32(float("-inf"))
    MRW = 256

    def init_half(lo, idx_ref):
        @pl.loop(lo, lo + HN)
        def _init(v):
            o = v * N_PATH
            ol = o - lo * N_PATH
            start = plsc.load_gather(nodes_v, [jnp.zeros((16,), jnp.int32) + v])
            path_v[pl.ds(o, 16)] = start
            score_v[pl.ds(o, 16)] = plsc.load_gather(cent_v, [start])
            ch = choices_v[0, pl.ds(o, 16)]
            idx_ref[pl.ds(ol, 16)] = ch * n_node + start

    def process_half(lo, nxt_ref, idx_ref, t, off):
        @pl.loop(lo, lo + HN)
        def _step(v):
            o = v * N_PATH
            ol = o - lo * N_PATH
            nxt = nxt_ref[pl.ds(ol, 16)]
            mr = plsc.load_gather(mr_v, [iota16, off + v])
            keep = mr >= t
            path_v[pl.ds(t * W + o, 16)] = jnp.where(keep, nxt, -1)
            c = plsc.load_gather(cent_v, [nxt])
            plsc.addupdate(score_v.at[pl.ds(o, 16)],
                           jnp.where(keep, c, jnp.float32(0.0)))
            if t < L_PATH - 1:
                ch = choices_v[t, pl.ds(o, 16)]
                idx_ref[pl.ds(ol, 16)] = ch * n_node + nxt

    OUTSZ = K_PATH * L_PATH * NT

    def do_tile(tid, parity):
        base = jnp.minimum(tid * NT, n_node - NT)
        wbase = pl.multiple_of(base * N_PATH, 128)
        base0 = pl.multiple_of(base - lax.rem(base, 128), 128)
        off = (base - base0) + jnp.zeros((16,), jnp.int32)
        pltpu.sync_copy(nodes_h.at[pl.ds(base, NT)], nodes_v)
        cp_ch = pltpu.async_copy(choices_h.at[:, pl.ds(wbase, W)],
                                 choices_v, sem_c)
        cp_mr = pltpu.async_copy(mr_h.at[:, pl.ds(base0, MRW)], mr_v, sem_c)
        cp_ch.wait()
        cp_mr.wait()

        init_half(0, idx_a)
        cp_a = pltpu.async_copy(neigh_s.at[idx_a], nxt_a, sem_a)
        init_half(HN, idx_b)
        cp_b = pltpu.async_copy(neigh_s.at[idx_b], nxt_b, sem_b)

        for t in range(1, L_PATH):
            cp_a.wait()
            process_half(0, nxt_a, idx_a, t, off)
            if t < L_PATH - 1:
                cp_a = pltpu.async_copy(neigh_s.at[idx_a], nxt_a, sem_a)
            cp_b.wait()
            process_half(HN, nxt_b, idx_b, t, off)
            if t < L_PATH - 1:
                cp_b = pltpu.async_copy(neigh_s.at[idx_b], nxt_b, sem_b)

        poff = parity * OUTSZ

        @pl.loop(0, NT)
        def _select(v):
            o = v * N_PATH
            s = score_v[pl.ds(o, 16)]
            picks = []
            for _ in range(K_PATH):
                m = s
                for sh in (1, 2, 4, 8):
                    m = jnp.maximum(m, _vshuf(m, iota ^ sh))
                i = plsc.all_reduce_ffs(s == m)
                picks.append(i)
                s = jnp.where(iota == i, neg_inf, s)
            sel_a = jnp.where(lane_lo, picks[0], picks[1])
            pa = plsc.load_gather(path_v, [iota8w + o + sel_a])
            plsc.store_scatter(out_v, [poff + kofn_a + v], pa)
            sel_b = jnp.where(lane_lo, picks[2], picks[3])
            pb = plsc.load_gather(path_v, [iota8w + o + sel_b])
            plsc.store_scatter(out_v, [poff + kofn_b + v], pb)

        @pl.when(parity == 0)
        def _():
            for seg in range(K_PATH * L_PATH):
                pltpu.async_copy(out_v.at[pl.ds(seg * NT, NT)],
                                 out_h.at[pl.ds(seg * n_node + base, NT)],
                                 sem_p)

        @pl.when(parity == 1)
        def _():
            for seg in range(K_PATH * L_PATH):
                pltpu.async_copy(out_v.at[pl.ds(OUTSZ + seg * NT, NT)],
                                 out_h.at[pl.ds(seg * n_node + base, NT)],
                                 sem_o)

    @pl.loop(0, nper)
    def _tiles(j):
        tid = wid + j * NWORKERS
        parity = lax.rem(j, 2)

        @pl.when(tid < ntiles)
        def _():
            @pl.when((j > 1) & (parity == 0))
            def _():
                pltpu.make_async_copy(out_h.at[pl.ds(0, OUTSZ)],
                                      out_v.at[pl.ds(0, OUTSZ)], sem_p).wait()

            @pl.when((j > 1) & (parity == 1))
            def _():
                pltpu.make_async_copy(out_h.at[pl.ds(0, OUTSZ)],
                                      out_v.at[pl.ds(OUTSZ, OUTSZ)],
                                      sem_o).wait()

            do_tile(tid, parity)

    pltpu.make_async_copy(out_h.at[pl.ds(0, OUTSZ)],
                          out_v.at[pl.ds(0, OUTSZ)], sem_p).wait()
    pltpu.make_async_copy(out_h.at[pl.ds(0, OUTSZ)],
                          out_v.at[pl.ds(OUTSZ, OUTSZ)], sem_o).wait()


def kernel(nodes, neighbors, centrality, walk_choices, mask_rand):
    n_node = nodes.shape[0]
    ntiles = -(-n_node // NT)
    nper = -(-ntiles // NWORKERS)
    max_base0 = (n_node - NT) - ((n_node - NT) % 128)
    mr_pad = max_base0 + 256 - n_node

    mesh = plsc.VectorSubcoreMesh(core_axis_name="c", subcore_axis_name="s")
    run = pl.kernel(
        functools.partial(_sampler_body, n_node, ntiles, nper),
        out_type=jax.ShapeDtypeStruct((n_node * K_PATH * L_PATH,), jnp.int32),
        mesh=mesh,
        compiler_params=pltpu.CompilerParams(needs_layout_passes=False),
        scratch_types=[
            pltpu.VMEM_SHARED((n_node * DEG,), jnp.int32),
            pltpu.VMEM((n_node,), jnp.float32),
            pltpu.VMEM((NT,), jnp.int32),
            pltpu.VMEM((L_PATH - 1, W), jnp.int32),
            pltpu.VMEM((N_PATH, 256), jnp.int32),
            pltpu.VMEM((HW,), jnp.int32),
            pltpu.VMEM((HW,), jnp.int32),
            pltpu.VMEM((HW,), jnp.int32),
            pltpu.VMEM((HW,), jnp.int32),
            pltpu.VMEM((L_PATH * W,), jnp.int32),
            pltpu.VMEM((W,), jnp.float32),
            pltpu.VMEM((2 * K_PATH * L_PATH * NT,), jnp.int32),
            pltpu.VMEM((2000,), jnp.int32),
            pltpu.SemaphoreType.DMA,
            pltpu.SemaphoreType.DMA,
            pltpu.SemaphoreType.DMA,
            pltpu.SemaphoreType.DMA,
            pltpu.SemaphoreType.DMA,
        ],
    )
    out = run(nodes,
              jnp.swapaxes(neighbors, 0, 1).reshape(-1),
              centrality,
              jnp.swapaxes(walk_choices, 0, 1),
              jnp.pad(jnp.swapaxes(mask_rand, 0, 1), ((0, 0), (0, mr_pad))))
    return jnp.transpose(out.reshape(K_PATH, L_PATH, n_node), (2, 0, 1))
